# Initial kernel scaffold; baseline (speedup 1.0000x reference)
import jax
import jax.numpy as jnp
from jax.experimental import pallas as pl


def kernel(X, vertex, edges, W, att_e):
    raise NotImplementedError("write your pallas kernel here")


# speedup vs baseline: 5.1220x; 5.1220x over previous
import functools

import jax
import jax.numpy as jnp
from jax import lax
from jax.experimental import pallas as pl
from jax.experimental.pallas import tpu as pltpu
from jax.experimental.pallas import tpu_sc as plsc

N = 10000
NNZ = 320000
E = 20000
IN = 128
H = 8
C = 16
HC = H * C
NEG_SLOPE = 0.2

NC = 2
NS = 16
CH = 80

EH = E // NC
TRASH = EH
AROWS = 10240
CBASE = AROWS
CROWS = 1280
TOTA = AROWS + CROWS
TPB = 624

EPACK = E // 8

VH = N // NC
VTRASH = VH
DBASE = 5120
TOTB = 2 * DBASE
VPB = 312

_mesh = plsc.VectorSubcoreMesh(core_axis_name="c", subcore_axis_name="s")


def _mm_body(x_ref, w_ref, o_ref):
    o_ref[...] = jnp.dot(x_ref[...], w_ref[...],
                         preferred_element_type=jnp.float32)


def _matmul(X, W):
    return pl.pallas_call(
        _mm_body,
        grid=(10,),
        in_specs=[
            pl.BlockSpec((N // 10, IN), lambda i: (i, 0)),
            pl.BlockSpec((IN, HC), lambda i: (0, 0)),
        ],
        out_specs=pl.BlockSpec((N // 10, HC), lambda i: (i, 0)),
        out_shape=jax.ShapeDtypeStruct((N, HC), jnp.float32),
    )(X, W)


@functools.partial(
    pl.kernel,
    out_type=(jax.ShapeDtypeStruct((E, HC), jnp.float32),
              jax.ShapeDtypeStruct((NC * CROWS, HC), jnp.float32)),
    mesh=_mesh,
    scratch_types=[
        pltpu.VMEM((CH,), jnp.int32),
        pltpu.VMEM((CH,), jnp.int32),
        pltpu.VMEM((CH,), jnp.int32),
        pltpu.VMEM((CH,), jnp.int32),
        pltpu.VMEM((CH, HC), jnp.float32),
        pltpu.VMEM((CH, HC), jnp.float32),
        pltpu.VMEM_SHARED((TOTA, HC), jnp.float32),
        pltpu.SemaphoreType.DMA,
    ],
)
def _pass_a(xh_hbm, vtx_hbm, edg_hbm, po_hbm, xesum_hbm, cnt_hbm,
            vid_v, eloc_v, ecc_v, pix_v, rows_v, pbuf_v, acc_sh, sem):
    c = lax.axis_index("c")
    s = lax.axis_index("s")
    lo = c * EH

    z16 = jnp.zeros((16,), jnp.float32)

    def _zb(i, _):
        for j in range(HC // 16):
            rows_v[i, pl.ds(j * 16, 16)] = z16
        return 0
    lax.fori_loop(0, CH, _zb, 0)

    zbase = s * (TOTA // NS)
    def _za(k, _):
        pltpu.sync_copy(rows_v, acc_sh.at[pl.ds(zbase + k * CH, CH)])
        return 0
    lax.fori_loop(0, (TOTA // NS) // CH, _za, 0)

    plsc.subcore_barrier()

    share = NNZ // NS
    base_i = s * share

    def _chunk(k, _):
        off = base_i + k * CH
        pltpu.sync_copy(vtx_hbm.at[pl.ds(off, CH)], vid_v)
        pltpu.sync_copy(edg_hbm.at[pl.ds(off, CH)], eloc_v)
        for j in range(CH // 16):
            e = eloc_v[pl.ds(j * 16, 16)]
            le = e - lo
            ok = (le >= 0) & (le < EH)
            le = jnp.where(ok, le, TRASH)
            eloc_v[pl.ds(j * 16, 16)] = le
            ecc_v[pl.ds(j * 16, 16)] = CBASE + (le >> 3)
            pix_v[pl.ds(j * 16, 16)] = le & 7
        pltpu.async_copy(xh_hbm.at[vid_v], rows_v, sem).wait()
        pltpu.sync_copy(rows_v, acc_sh.at[eloc_v], add=True)
        pltpu.async_copy(po_hbm.at[pix_v], pbuf_v, sem).wait()
        pltpu.sync_copy(pbuf_v, acc_sh.at[ecc_v], add=True)
        return 0
    lax.fori_loop(0, share // CH, _chunk, 0)

    plsc.subcore_barrier()

    ob = s * TPB
    pltpu.sync_copy(acc_sh.at[pl.ds(ob, TPB)],
                    xesum_hbm.at[pl.ds(lo + ob, TPB)])

    @pl.when(s == 0)
    def _():
        pltpu.sync_copy(acc_sh.at[pl.ds(NS * TPB, EH - NS * TPB)],
                        xesum_hbm.at[pl.ds(lo + NS * TPB, EH - NS * TPB)])

    pltpu.sync_copy(acc_sh.at[pl.ds(CBASE + s * CH, CH)],
                    cnt_hbm.at[pl.ds(c * CROWS + s * CH, CH)])


_EB = 2000


def _k3a_body(xesum_ref, cnt_ref, a_ref, xe_ref, k_ref, kacc):
    i = pl.program_id(0)
    cnt = jnp.broadcast_to(jnp.maximum(cnt_ref[...], 1.0), (_EB, HC))
    xe = xesum_ref[...] / cnt
    xe_ref[...] = xe
    alpha = jnp.dot(xe, a_ref[...], preferred_element_type=jnp.float32)
    al = jnp.where(alpha >= 0, alpha, NEG_SLOPE * alpha)
    bmax = jnp.max(al, axis=0, keepdims=True)

    @pl.when(i == 0)
    def _():
        kacc[...] = bmax

    @pl.when(i > 0)
    def _():
        kacc[...] = jnp.maximum(kacc[...], bmax)

    k_ref[...] = kacc[...]


def _k3a(Xe_sum, cnt_e, A):
    return pl.pallas_call(
        _k3a_body,
        grid=(E // _EB,),
        in_specs=[
            pl.BlockSpec((_EB, HC), lambda i: (i, 0)),
            pl.BlockSpec((_EB, 1), lambda i: (i, 0)),
            pl.BlockSpec((HC, H), lambda i: (0, 0)),
        ],
        out_specs=[
            pl.BlockSpec((_EB, HC), lambda i: (i, 0)),
            pl.BlockSpec((1, H), lambda i: (0, 0)),
        ],
        out_shape=[
            jax.ShapeDtypeStruct((E, HC), jnp.float32),
            jax.ShapeDtypeStruct((1, H), jnp.float32),
        ],
        scratch_shapes=[pltpu.VMEM((1, H), jnp.float32)],
    )(Xe_sum, cnt_e, A)


def _k3b_body(xe_ref, a_ref, k_ref, t_ref, g_ref):
    xe = xe_ref[...]
    alpha = jnp.dot(xe, a_ref[...], preferred_element_type=jnp.float32)
    al = jnp.where(alpha >= 0, alpha, NEG_SLOPE * alpha)
    g = jnp.exp(al - k_ref[...])
    gexp = jnp.broadcast_to(g[:, :, None], (_EB, H, C)).reshape(_EB, HC)
    t_ref[...] = xe * gexp
    g_ref[...] = gexp


def _k3b(Xe, A, K):
    return pl.pallas_call(
        _k3b_body,
        grid=(E // _EB,),
        in_specs=[
            pl.BlockSpec((_EB, HC), lambda i: (i, 0)),
            pl.BlockSpec((HC, H), lambda i: (0, 0)),
            pl.BlockSpec((1, H), lambda i: (0, 0)),
        ],
        out_specs=[
            pl.BlockSpec((_EB, HC), lambda i: (i, 0)),
            pl.BlockSpec((_EB, HC), lambda i: (i, 0)),
        ],
        out_shape=[
            jax.ShapeDtypeStruct((E, HC), jnp.float32),
            jax.ShapeDtypeStruct((E, HC), jnp.float32),
        ],
    )(Xe, A, K)


@functools.partial(
    pl.kernel,
    out_type=(jax.ShapeDtypeStruct((N, HC), jnp.float32),
              jax.ShapeDtypeStruct((N, HC), jnp.float32)),
    mesh=_mesh,
    scratch_types=[
        pltpu.VMEM((CH,), jnp.int32),
        pltpu.VMEM((CH,), jnp.int32),
        pltpu.VMEM((CH,), jnp.int32),
        pltpu.VMEM((CH, HC), jnp.float32),
        pltpu.VMEM((CH, HC), jnp.float32),
        pltpu.VMEM_SHARED((TOTB, HC), jnp.float32),
        pltpu.SemaphoreType.DMA,
    ],
)
def _pass_b(t_hbm, g_hbm, vtx_hbm, edg_hbm, p_hbm, s_hbm,
            eid_v, vid_v, dcc_v, trows_v, grows_v, acc_sh, sem):
    c = lax.axis_index("c")
    s = lax.axis_index("s")
    lo = c * VH

    z16 = jnp.zeros((16,), jnp.float32)

    def _zb(i, _):
        for j in range(HC // 16):
            trows_v[i, pl.ds(j * 16, 16)] = z16
        return 0
    lax.fori_loop(0, CH, _zb, 0)

    zbase = s * (TOTB // NS)
    def _za(k, _):
        pltpu.sync_copy(trows_v, acc_sh.at[pl.ds(zbase + k * CH, CH)])
        return 0
    lax.fori_loop(0, (TOTB // NS) // CH, _za, 0)

    plsc.subcore_barrier()

    share = NNZ // NS
    base_i = s * share

    def _chunk(k, _):
        off = base_i + k * CH
        pltpu.sync_copy(edg_hbm.at[pl.ds(off, CH)], eid_v)
        pltpu.sync_copy(vtx_hbm.at[pl.ds(off, CH)], vid_v)
        for j in range(CH // 16):
            v = vid_v[pl.ds(j * 16, 16)]
            lv = v - lo
            ok = (lv >= 0) & (lv < VH)
            lv = jnp.where(ok, lv, VTRASH)
            vid_v[pl.ds(j * 16, 16)] = lv
            dcc_v[pl.ds(j * 16, 16)] = DBASE + lv
        pltpu.async_copy(t_hbm.at[eid_v], trows_v, sem).wait()
        pltpu.sync_copy(trows_v, acc_sh.at[vid_v], add=True)
        pltpu.async_copy(g_hbm.at[eid_v], grows_v, sem).wait()
        pltpu.sync_copy(grows_v, acc_sh.at[dcc_v], add=True)
        return 0
    lax.fori_loop(0, share // CH, _chunk, 0)

    plsc.subcore_barrier()

    ob = s * VPB
    pltpu.sync_copy(acc_sh.at[pl.ds(ob, VPB)],
                    p_hbm.at[pl.ds(lo + ob, VPB)])
    pltpu.sync_copy(acc_sh.at[pl.ds(DBASE + ob, VPB)],
                    s_hbm.at[pl.ds(lo + ob, VPB)])

    @pl.when(s == 0)
    def _():
        pltpu.sync_copy(acc_sh.at[pl.ds(NS * VPB, VH - NS * VPB)],
                        p_hbm.at[pl.ds(lo + NS * VPB, VH - NS * VPB)])
        pltpu.sync_copy(acc_sh.at[pl.ds(DBASE + NS * VPB, VH - NS * VPB)],
                        s_hbm.at[pl.ds(lo + NS * VPB, VH - NS * VPB)])


_NB = 2000


def _k5_body(p_ref, s_ref, o_ref):
    xv = p_ref[...] / (s_ref[...] + 1e-16)
    o_ref[...] = xv * 0.5 * (1.0 + lax.erf(xv * (2.0 ** -0.5)))


def _k5(P, S):
    return pl.pallas_call(
        _k5_body,
        grid=(N // _NB,),
        in_specs=[
            pl.BlockSpec((_NB, HC), lambda i: (i, 0)),
            pl.BlockSpec((_NB, HC), lambda i: (i, 0)),
        ],
        out_specs=pl.BlockSpec((_NB, HC), lambda i: (i, 0)),
        out_shape=jax.ShapeDtypeStruct((N, HC), jnp.float32),
    )(P, S)


def kernel(X, vertex, edges, W, att_e):
    Xh = _matmul(X, W)
    pones = jnp.zeros((8, HC), jnp.float32).at[
        jnp.arange(8), jnp.arange(8) * 16].set(1.0)
    Xe_sum, cntp = _pass_a(Xh, vertex, edges, pones)
    att = att_e.reshape(H, C)
    A = (jnp.eye(H, dtype=jnp.float32)[:, None, :] * att[:, :, None]
         ).reshape(HC, H)
    cntp2 = jnp.concatenate([cntp[0:EPACK // 2], cntp[CROWS:CROWS + EPACK // 2]])
    cnt_e = cntp2.reshape(EPACK, 8, 16)[:, :, 0].reshape(E, 1)
    Xe, K = _k3a(Xe_sum, cnt_e, A)
    T, G = _k3b(Xe, A, K)
    P, S = _pass_b(T, G, vertex, edges)
    return _k5(P, S)
